# initial kernel scaffold (unmeasured)
import jax
import jax.numpy as jnp
from jax import lax
from jax.experimental import pallas as pl
from jax.experimental.pallas import tpu as pltpu

N_DEV = 32
M_BLK = 1024 // N_DEV
K_BLK = 1024 // N_DEV


def kernel(x, w_mat):
    m_glob, k_per = x.shape
    k_glob, n = w_mat.shape

    def body(x_ref, w_ref, out_ref, send_buf, recv_buf, send_sems, recv_sems):
        me = lax.axis_index("i")

        send_buf[...] = x_ref[...].astype(jnp.bfloat16).reshape(N_DEV, M_BLK, K_BLK)

        recv_buf[pl.ds(me, 1)] = send_buf[pl.ds(me, 1)]

        for s in range(1, N_DEV):
            dst = (me + s) % N_DEV
            rdma = pltpu.make_async_remote_copy(
                src_ref=send_buf.at[dst],
                dst_ref=recv_buf.at[me],
                send_sem=send_sems.at[dst],
                recv_sem=recv_sems.at[me],
                device_id=(dst,),
                device_id_type=pl.DeviceIdType.MESH,
            )
            rdma.start()

        for s in range(1, N_DEV):
            src = (me + s) % N_DEV
            recv = pltpu.make_async_remote_copy(
                src_ref=send_buf.at[src],
                dst_ref=recv_buf.at[src],
                send_sem=send_sems.at[src],
                recv_sem=recv_sems.at[src],
                device_id=(src,),
                device_id_type=pl.DeviceIdType.MESH,
            )
            recv.wait_recv()

        xall = recv_buf[...]
        xrows = xall.transpose(1, 0, 2).reshape(M_BLK, k_glob)
        w = w_ref[...].astype(jnp.bfloat16)
        y = jnp.dot(xrows, w, preferred_element_type=jnp.float32)
        out_ref[...] = jnp.maximum(y, 0.0)

        for s in range(1, N_DEV):
            dst = (me + s) % N_DEV
            snd = pltpu.make_async_remote_copy(
                src_ref=send_buf.at[dst],
                dst_ref=recv_buf.at[me],
                send_sem=send_sems.at[dst],
                recv_sem=recv_sems.at[me],
                device_id=(dst,),
                device_id_type=pl.DeviceIdType.MESH,
            )
            snd.wait_send()

    return pl.pallas_call(
        body,
        out_shape=jax.ShapeDtypeStruct((M_BLK, n), jnp.float32),
        in_specs=[
            pl.BlockSpec(memory_space=pltpu.VMEM),
            pl.BlockSpec(memory_space=pltpu.VMEM),
        ],
        out_specs=pl.BlockSpec(memory_space=pltpu.VMEM),
        scratch_shapes=[
            pltpu.VMEM((N_DEV, M_BLK, K_BLK), jnp.bfloat16),
            pltpu.VMEM((N_DEV, M_BLK, K_BLK), jnp.bfloat16),
            pltpu.SemaphoreType.DMA((N_DEV,)),
            pltpu.SemaphoreType.DMA((N_DEV,)),
        ],
        compiler_params=pltpu.CompilerParams(collective_id=0),
    )(x, w_mat)


# baseline (device time: 23529 ns/iter reference)
import jax
import jax.numpy as jnp
from jax import lax
from jax.experimental import pallas as pl
from jax.experimental.pallas import tpu as pltpu

N_DEV = 32
M_BLK = 1024 // N_DEV
K_BLK = 1024 // N_DEV


def kernel(x, w_mat):
    m_glob, k_per = x.shape
    k_glob, n = w_mat.shape

    def body(x_ref, w_ref, out_ref, send_buf, recv_buf, send_sems, recv_sems):
        me = lax.axis_index("i")

        send_buf[...] = x_ref[...].astype(jnp.bfloat16).reshape(N_DEV, M_BLK, K_BLK)

        recv_buf[pl.ds(me, 1)] = send_buf[pl.ds(me, 1)]

        for s in range(1, N_DEV):
            dst = (me + s) % N_DEV
            rdma = pltpu.make_async_remote_copy(
                src_ref=send_buf.at[dst],
                dst_ref=recv_buf.at[me],
                send_sem=send_sems.at[dst],
                recv_sem=recv_sems.at[me],
                device_id=(dst,),
                device_id_type=pl.DeviceIdType.MESH,
            )
            rdma.start()

        for s in range(1, N_DEV):
            src = (me + s) % N_DEV
            recv = pltpu.make_async_remote_copy(
                src_ref=send_buf.at[src],
                dst_ref=recv_buf.at[src],
                send_sem=send_sems.at[src],
                recv_sem=recv_sems.at[src],
                device_id=(src,),
                device_id_type=pl.DeviceIdType.MESH,
            )
            recv.wait_recv()

        xall = recv_buf[...]
        xrows = xall.transpose(1, 0, 2).reshape(M_BLK, k_glob)
        w = w_ref[...].astype(jnp.bfloat16)
        y = jnp.dot(xrows, w, preferred_element_type=jnp.float32)
        out_ref[...] = jnp.maximum(y, 0.0)

        for s in range(1, N_DEV):
            dst = (me + s) % N_DEV
            snd = pltpu.make_async_remote_copy(
                src_ref=send_buf.at[dst],
                dst_ref=recv_buf.at[me],
                send_sem=send_sems.at[dst],
                recv_sem=recv_sems.at[me],
                device_id=(dst,),
                device_id_type=pl.DeviceIdType.MESH,
            )
            snd.wait_send()

    return pl.pallas_call(
        body,
        out_shape=jax.ShapeDtypeStruct((M_BLK, n), jnp.float32),
        in_specs=[
            pl.BlockSpec(memory_space=pltpu.VMEM),
            pl.BlockSpec(memory_space=pltpu.VMEM),
        ],
        out_specs=pl.BlockSpec(memory_space=pltpu.VMEM),
        scratch_shapes=[
            pltpu.VMEM((N_DEV, M_BLK, K_BLK), jnp.bfloat16),
            pltpu.VMEM((N_DEV, M_BLK, K_BLK), jnp.bfloat16),
            pltpu.SemaphoreType.DMA((N_DEV,)),
            pltpu.SemaphoreType.DMA((N_DEV,)),
        ],
    )(x, w_mat)


# device time: 18822 ns/iter; 1.2501x vs baseline; 1.2501x over previous
import jax
import jax.numpy as jnp
from jax import lax
from jax.experimental import pallas as pl
from jax.experimental.pallas import tpu as pltpu

N_DEV = 32
NZ = 4
NQ = 8
M_BLK = 1024 // N_DEV
K_BLK = 1024 // N_DEV


def kernel(x, w_mat):
    m_glob, k_per = x.shape
    k_glob, n = w_mat.shape

    def body(x_ref, w_hbm, out_ref, send_buf, recv_a, recv_b, w_vmem,
             send_a_sems, recv_a_sems, send_b_sems, recv_b_sems, w_sem):
        me = lax.axis_index("i")
        my_z = me // NQ
        my_q = me % NQ

        w_copy = pltpu.make_async_copy(w_hbm, w_vmem, w_sem)
        w_copy.start()

        send_buf[...] = x_ref[...].astype(jnp.bfloat16).reshape(NZ, NQ, M_BLK, K_BLK)

        bar = pltpu.get_barrier_semaphore()
        for off in range(1, NZ):
            tz = (my_z + off) % NZ
            pl.semaphore_signal(bar, inc=1, device_id=(tz * NQ + my_q,),
                                device_id_type=pl.DeviceIdType.MESH)
        for off in range(1, NQ):
            tq = (my_q + off) % NQ
            pl.semaphore_signal(bar, inc=1, device_id=(my_z * NQ + tq,),
                                device_id_type=pl.DeviceIdType.MESH)
        pl.semaphore_wait(bar, (NZ - 1) + (NQ - 1))

        for off in range(1, NZ):
            tz = (my_z + off) % NZ
            pltpu.make_async_remote_copy(
                src_ref=send_buf.at[pl.ds(tz, 1)],
                dst_ref=recv_a.at[pl.ds(my_z, 1)],
                send_sem=send_a_sems.at[tz],
                recv_sem=recv_a_sems.at[my_z],
                device_id=(tz * NQ + my_q,),
                device_id_type=pl.DeviceIdType.MESH,
            ).start()
        recv_a[pl.ds(my_z, 1)] = send_buf[pl.ds(my_z, 1)]
        for off in range(1, NZ):
            sz = (my_z + off) % NZ
            pltpu.make_async_remote_copy(
                src_ref=send_buf.at[pl.ds(sz, 1)],
                dst_ref=recv_a.at[pl.ds(sz, 1)],
                send_sem=send_a_sems.at[sz],
                recv_sem=recv_a_sems.at[sz],
                device_id=(sz * NQ + my_q,),
                device_id_type=pl.DeviceIdType.MESH,
            ).wait_recv()

        for off in range(1, NQ):
            tq = (my_q + off) % NQ
            pltpu.make_async_remote_copy(
                src_ref=recv_a.at[:, pl.ds(tq, 1)],
                dst_ref=recv_b.at[:, pl.ds(my_q, 1)],
                send_sem=send_b_sems.at[tq],
                recv_sem=recv_b_sems.at[my_q],
                device_id=(my_z * NQ + tq,),
                device_id_type=pl.DeviceIdType.MESH,
            ).start()
        recv_b[:, pl.ds(my_q, 1)] = recv_a[:, pl.ds(my_q, 1)]
        for off in range(1, NQ):
            sq = (my_q + off) % NQ
            pltpu.make_async_remote_copy(
                src_ref=recv_a.at[:, pl.ds(sq, 1)],
                dst_ref=recv_b.at[:, pl.ds(sq, 1)],
                send_sem=send_b_sems.at[sq],
                recv_sem=recv_b_sems.at[sq],
                device_id=(my_z * NQ + sq,),
                device_id_type=pl.DeviceIdType.MESH,
            ).wait_recv()

        xrows = recv_b[...].transpose(2, 0, 1, 3).reshape(M_BLK, k_glob)
        w_copy.wait()
        y = jnp.dot(xrows.astype(jnp.float32), w_vmem[...],
                    preferred_element_type=jnp.float32)
        out_ref[...] = jnp.maximum(y, 0.0)

        for off in range(1, NZ):
            tz = (my_z + off) % NZ
            pltpu.make_async_remote_copy(
                src_ref=send_buf.at[pl.ds(tz, 1)],
                dst_ref=recv_a.at[pl.ds(my_z, 1)],
                send_sem=send_a_sems.at[tz],
                recv_sem=recv_a_sems.at[my_z],
                device_id=(tz * NQ + my_q,),
                device_id_type=pl.DeviceIdType.MESH,
            ).wait_send()
        for off in range(1, NQ):
            tq = (my_q + off) % NQ
            pltpu.make_async_remote_copy(
                src_ref=recv_a.at[:, pl.ds(tq, 1)],
                dst_ref=recv_b.at[:, pl.ds(my_q, 1)],
                send_sem=send_b_sems.at[tq],
                recv_sem=recv_b_sems.at[my_q],
                device_id=(my_z * NQ + tq,),
                device_id_type=pl.DeviceIdType.MESH,
            ).wait_send()

    return pl.pallas_call(
        body,
        out_shape=jax.ShapeDtypeStruct((M_BLK, n), jnp.float32),
        in_specs=[
            pl.BlockSpec(memory_space=pltpu.VMEM),
            pl.BlockSpec(memory_space=pl.ANY),
        ],
        out_specs=pl.BlockSpec(memory_space=pltpu.VMEM),
        scratch_shapes=[
            pltpu.VMEM((NZ, NQ, M_BLK, K_BLK), jnp.bfloat16),
            pltpu.VMEM((NZ, NQ, M_BLK, K_BLK), jnp.bfloat16),
            pltpu.VMEM((NZ, NQ, M_BLK, K_BLK), jnp.bfloat16),
            pltpu.VMEM((1024, 1024), jnp.float32),
            pltpu.SemaphoreType.DMA((NZ,)),
            pltpu.SemaphoreType.DMA((NZ,)),
            pltpu.SemaphoreType.DMA((NQ,)),
            pltpu.SemaphoreType.DMA((NQ,)),
            pltpu.SemaphoreType.DMA,
        ],
        compiler_params=pltpu.CompilerParams(collective_id=0),
    )(x, w_mat)


# device time: 18557 ns/iter; 1.2679x vs baseline; 1.0143x over previous
import jax
import jax.numpy as jnp
from jax import lax
from jax.experimental import pallas as pl
from jax.experimental.pallas import tpu as pltpu

N_DEV = 32
NZ = 4
NQ = 8
M_BLK = 1024 // N_DEV
K_BLK = 1024 // N_DEV


def kernel(x, w_mat):
    m_glob, k_per = x.shape
    k_glob, n = w_mat.shape

    def body(x_ref, w_hbm, out_ref, send_buf, recv_a, recv_b, w_vmem,
             send_a_sems, recv_a_sems, send_b_sems, recv_b_sems, w_sem,
             ready_sem):
        me = lax.axis_index("i")
        my_z = me // NQ
        my_q = me % NQ

        w_copy = pltpu.make_async_copy(w_hbm, w_vmem, w_sem)
        w_copy.start()

        bar = pltpu.get_barrier_semaphore()
        for off in range(1, NZ):
            tz = (my_z + off) % NZ
            pl.semaphore_signal(bar, inc=1, device_id=(tz * NQ + my_q,),
                                device_id_type=pl.DeviceIdType.MESH)

        send_buf[...] = x_ref[...].astype(jnp.bfloat16).reshape(NZ, NQ, M_BLK, K_BLK)

        pl.semaphore_wait(bar, NZ - 1)

        for off in range(1, NZ):
            tz = (my_z + off) % NZ
            pltpu.make_async_remote_copy(
                src_ref=send_buf.at[pl.ds(tz, 1)],
                dst_ref=recv_a.at[pl.ds(my_z, 1)],
                send_sem=send_a_sems.at[tz],
                recv_sem=recv_a_sems.at[my_z],
                device_id=(tz * NQ + my_q,),
                device_id_type=pl.DeviceIdType.MESH,
            ).start()

        for off in range(1, NQ):
            tq = (my_q + off) % NQ
            pl.semaphore_signal(ready_sem, inc=1, device_id=(my_z * NQ + tq,),
                                device_id_type=pl.DeviceIdType.MESH)

        recv_a[pl.ds(my_z, 1)] = send_buf[pl.ds(my_z, 1)]

        for off in range(1, NZ):
            sz = (my_z + off) % NZ
            pltpu.make_async_remote_copy(
                src_ref=send_buf.at[pl.ds(sz, 1)],
                dst_ref=recv_a.at[pl.ds(sz, 1)],
                send_sem=send_a_sems.at[sz],
                recv_sem=recv_a_sems.at[sz],
                device_id=(sz * NQ + my_q,),
                device_id_type=pl.DeviceIdType.MESH,
            ).wait_recv()
        pl.semaphore_wait(ready_sem, NQ - 1)

        for off in range(1, NQ):
            tq = (my_q + off) % NQ
            pltpu.make_async_remote_copy(
                src_ref=recv_a.at[:, pl.ds(tq, 1)],
                dst_ref=recv_b.at[:, pl.ds(my_q, 1)],
                send_sem=send_b_sems.at[tq],
                recv_sem=recv_b_sems.at[my_q],
                device_id=(my_z * NQ + tq,),
                device_id_type=pl.DeviceIdType.MESH,
            ).start()
        recv_b[:, pl.ds(my_q, 1)] = recv_a[:, pl.ds(my_q, 1)]
        for off in range(1, NQ):
            sq = (my_q + off) % NQ
            pltpu.make_async_remote_copy(
                src_ref=recv_a.at[:, pl.ds(sq, 1)],
                dst_ref=recv_b.at[:, pl.ds(sq, 1)],
                send_sem=send_b_sems.at[sq],
                recv_sem=recv_b_sems.at[sq],
                device_id=(my_z * NQ + sq,),
                device_id_type=pl.DeviceIdType.MESH,
            ).wait_recv()

        xrows = recv_b[...].transpose(2, 0, 1, 3).reshape(M_BLK, k_glob)
        w_copy.wait()
        y = jnp.dot(xrows.astype(jnp.float32), w_vmem[...],
                    preferred_element_type=jnp.float32)
        out_ref[...] = jnp.maximum(y, 0.0).astype(jnp.bfloat16)

        for off in range(1, NZ):
            tz = (my_z + off) % NZ
            pltpu.make_async_remote_copy(
                src_ref=send_buf.at[pl.ds(tz, 1)],
                dst_ref=recv_a.at[pl.ds(my_z, 1)],
                send_sem=send_a_sems.at[tz],
                recv_sem=recv_a_sems.at[my_z],
                device_id=(tz * NQ + my_q,),
                device_id_type=pl.DeviceIdType.MESH,
            ).wait_send()
        for off in range(1, NQ):
            tq = (my_q + off) % NQ
            pltpu.make_async_remote_copy(
                src_ref=recv_a.at[:, pl.ds(tq, 1)],
                dst_ref=recv_b.at[:, pl.ds(my_q, 1)],
                send_sem=send_b_sems.at[tq],
                recv_sem=recv_b_sems.at[my_q],
                device_id=(my_z * NQ + tq,),
                device_id_type=pl.DeviceIdType.MESH,
            ).wait_send()

    return pl.pallas_call(
        body,
        out_shape=jax.ShapeDtypeStruct((M_BLK, n), jnp.bfloat16),
        in_specs=[
            pl.BlockSpec(memory_space=pltpu.VMEM),
            pl.BlockSpec(memory_space=pl.ANY),
        ],
        out_specs=pl.BlockSpec(memory_space=pltpu.VMEM),
        scratch_shapes=[
            pltpu.VMEM((NZ, NQ, M_BLK, K_BLK), jnp.bfloat16),
            pltpu.VMEM((NZ, NQ, M_BLK, K_BLK), jnp.bfloat16),
            pltpu.VMEM((NZ, NQ, M_BLK, K_BLK), jnp.bfloat16),
            pltpu.VMEM((1024, 1024), jnp.float32),
            pltpu.SemaphoreType.DMA((NZ,)),
            pltpu.SemaphoreType.DMA((NZ,)),
            pltpu.SemaphoreType.DMA((NQ,)),
            pltpu.SemaphoreType.DMA((NQ,)),
            pltpu.SemaphoreType.DMA,
            pltpu.SemaphoreType.REGULAR,
        ],
        compiler_params=pltpu.CompilerParams(collective_id=0),
    )(x, w_mat)


# device time: 15724 ns/iter; 1.4964x vs baseline; 1.1802x over previous
import jax
import jax.numpy as jnp
from jax import lax
from jax.experimental import pallas as pl
from jax.experimental.pallas import tpu as pltpu

N_DEV = 32
NZ = 4
NQ = 8
M_BLK = 1024 // N_DEV
K_BLK = 1024 // N_DEV


def kernel(x, w_mat):
    m_glob, k_per = x.shape
    k_glob, n = w_mat.shape
    xt = x.T
    w_hbm = pltpu.with_memory_space_constraint(w_mat, pltpu.MemorySpace.HBM)

    def body(xt_ref, w_ref, out_ref, send_buf, recv_a, recv_b, w_vmem,
             send_a_sems, recv_a_sems, send_b_sems, recv_b_sems, w_sem,
             ready_sem):
        me = lax.axis_index("i")
        my_z = me // NQ
        my_q = me % NQ

        w_copy = pltpu.make_async_copy(w_ref, w_vmem, w_sem)
        w_copy.start()

        bar = pltpu.get_barrier_semaphore()
        for off in range(1, NZ):
            tz = (my_z + off) % NZ
            pl.semaphore_signal(bar, inc=1, device_id=(tz * NQ + my_q,),
                                device_id_type=pl.DeviceIdType.MESH)

        send_buf[...] = (
            xt_ref[...].astype(jnp.bfloat16)
            .reshape(K_BLK, NZ, NQ, M_BLK).transpose(1, 2, 0, 3)
        )

        pl.semaphore_wait(bar, NZ - 1)

        for off in range(1, NZ):
            tz = (my_z + off) % NZ
            pltpu.make_async_remote_copy(
                src_ref=send_buf.at[pl.ds(tz, 1)],
                dst_ref=recv_a.at[pl.ds(my_z, 1)],
                send_sem=send_a_sems.at[tz],
                recv_sem=recv_a_sems.at[my_z],
                device_id=(tz * NQ + my_q,),
                device_id_type=pl.DeviceIdType.MESH,
            ).start()

        for off in range(1, NQ):
            tq = (my_q + off) % NQ
            pl.semaphore_signal(ready_sem, inc=1, device_id=(my_z * NQ + tq,),
                                device_id_type=pl.DeviceIdType.MESH)

        recv_a[pl.ds(my_z, 1)] = send_buf[pl.ds(my_z, 1)]

        for off in range(1, NZ):
            sz = (my_z + off) % NZ
            pltpu.make_async_remote_copy(
                src_ref=send_buf.at[pl.ds(sz, 1)],
                dst_ref=recv_a.at[pl.ds(sz, 1)],
                send_sem=send_a_sems.at[sz],
                recv_sem=recv_a_sems.at[sz],
                device_id=(sz * NQ + my_q,),
                device_id_type=pl.DeviceIdType.MESH,
            ).wait_recv()
        pl.semaphore_wait(ready_sem, NQ - 1)

        for off in range(1, NQ):
            tq = (my_q + off) % NQ
            pltpu.make_async_remote_copy(
                src_ref=recv_a.at[:, pl.ds(tq, 1)],
                dst_ref=recv_b.at[:, pl.ds(my_q, 1)],
                send_sem=send_b_sems.at[tq],
                recv_sem=recv_b_sems.at[my_q],
                device_id=(my_z * NQ + tq,),
                device_id_type=pl.DeviceIdType.MESH,
            ).start()
        recv_b[:, pl.ds(my_q, 1)] = recv_a[:, pl.ds(my_q, 1)]
        for off in range(1, NQ):
            sq = (my_q + off) % NQ
            pltpu.make_async_remote_copy(
                src_ref=recv_a.at[:, pl.ds(sq, 1)],
                dst_ref=recv_b.at[:, pl.ds(sq, 1)],
                send_sem=send_b_sems.at[sq],
                recv_sem=recv_b_sems.at[sq],
                device_id=(my_z * NQ + sq,),
                device_id_type=pl.DeviceIdType.MESH,
            ).wait_recv()

        xkt = recv_b[...].reshape(k_glob, M_BLK).astype(jnp.float32)
        w_copy.wait()
        y = lax.dot_general(xkt, w_vmem[...], (((0,), (0,)), ((), ())),
                            preferred_element_type=jnp.float32)
        out_ref[...] = jnp.maximum(y, 0.0).astype(jnp.bfloat16)

        for off in range(1, NZ):
            tz = (my_z + off) % NZ
            pltpu.make_async_remote_copy(
                src_ref=send_buf.at[pl.ds(tz, 1)],
                dst_ref=recv_a.at[pl.ds(my_z, 1)],
                send_sem=send_a_sems.at[tz],
                recv_sem=recv_a_sems.at[my_z],
                device_id=(tz * NQ + my_q,),
                device_id_type=pl.DeviceIdType.MESH,
            ).wait_send()
        for off in range(1, NQ):
            tq = (my_q + off) % NQ
            pltpu.make_async_remote_copy(
                src_ref=recv_a.at[:, pl.ds(tq, 1)],
                dst_ref=recv_b.at[:, pl.ds(my_q, 1)],
                send_sem=send_b_sems.at[tq],
                recv_sem=recv_b_sems.at[my_q],
                device_id=(my_z * NQ + tq,),
                device_id_type=pl.DeviceIdType.MESH,
            ).wait_send()

    return pl.pallas_call(
        body,
        out_shape=jax.ShapeDtypeStruct((M_BLK, n), jnp.bfloat16),
        in_specs=[
            pl.BlockSpec(memory_space=pltpu.VMEM),
            pl.BlockSpec(memory_space=pltpu.MemorySpace.HBM),
        ],
        out_specs=pl.BlockSpec(memory_space=pltpu.VMEM),
        scratch_shapes=[
            pltpu.VMEM((NZ, NQ, K_BLK, M_BLK), jnp.bfloat16),
            pltpu.VMEM((NZ, NQ, K_BLK, M_BLK), jnp.bfloat16),
            pltpu.VMEM((NZ, NQ, K_BLK, M_BLK), jnp.bfloat16),
            pltpu.VMEM((1024, 1024), jnp.float32),
            pltpu.SemaphoreType.DMA((NZ,)),
            pltpu.SemaphoreType.DMA((NZ,)),
            pltpu.SemaphoreType.DMA((NQ,)),
            pltpu.SemaphoreType.DMA((NQ,)),
            pltpu.SemaphoreType.DMA,
            pltpu.SemaphoreType.REGULAR,
        ],
        compiler_params=pltpu.CompilerParams(collective_id=0),
    )(xt, w_hbm)


# device time: 14978 ns/iter; 1.5709x vs baseline; 1.0498x over previous
import jax
import jax.numpy as jnp
from jax import lax
from jax.experimental import pallas as pl
from jax.experimental.pallas import tpu as pltpu

N_DEV = 32
NZ = 4
NQ = 8
M_BLK = 1024 // N_DEV
K_BLK = 1024 // N_DEV


def kernel(x, w_mat):
    m_glob, k_per = x.shape
    k_glob, n = w_mat.shape
    xt = x.T
    xt = pltpu.with_memory_space_constraint(xt, pltpu.MemorySpace.HBM)
    w_hbm = pltpu.with_memory_space_constraint(w_mat, pltpu.MemorySpace.HBM)

    def body(xt_ref, w_ref, out_ref, send_buf, recv_a, recv_b, w_vmem,
             xt_vmem, y_vmem, send_a_sems, recv_a_sems, send_b_sems,
             recv_b_sems, w_sem, x_sem, out_sem, ready_sem):
        me = lax.axis_index("i")
        my_z = me // NQ
        my_q = me % NQ

        x_copy = pltpu.make_async_copy(xt_ref, xt_vmem, x_sem)
        x_copy.start()
        w_copy = pltpu.make_async_copy(w_ref, w_vmem, w_sem)
        w_copy.start()

        bar = pltpu.get_barrier_semaphore()
        for off in range(1, NZ):
            tz = (my_z + off) % NZ
            pl.semaphore_signal(bar, inc=1, device_id=(tz * NQ + my_q,),
                                device_id_type=pl.DeviceIdType.MESH)

        x_copy.wait()
        send_buf[...] = (
            xt_vmem[...].astype(jnp.bfloat16)
            .reshape(K_BLK, NZ, NQ, M_BLK).transpose(1, 2, 0, 3)
        )

        pl.semaphore_wait(bar, NZ - 1)

        for off in range(1, NZ):
            tz = (my_z + off) % NZ
            pltpu.make_async_remote_copy(
                src_ref=send_buf.at[pl.ds(tz, 1)],
                dst_ref=recv_a.at[pl.ds(my_z, 1)],
                send_sem=send_a_sems.at[tz],
                recv_sem=recv_a_sems.at[my_z],
                device_id=(tz * NQ + my_q,),
                device_id_type=pl.DeviceIdType.MESH,
            ).start()

        for off in range(1, NQ):
            tq = (my_q + off) % NQ
            pl.semaphore_signal(ready_sem, inc=1, device_id=(my_z * NQ + tq,),
                                device_id_type=pl.DeviceIdType.MESH)

        recv_a[pl.ds(my_z, 1)] = send_buf[pl.ds(my_z, 1)]

        for off in range(1, NZ):
            sz = (my_z + off) % NZ
            pltpu.make_async_remote_copy(
                src_ref=send_buf.at[pl.ds(sz, 1)],
                dst_ref=recv_a.at[pl.ds(sz, 1)],
                send_sem=send_a_sems.at[sz],
                recv_sem=recv_a_sems.at[sz],
                device_id=(sz * NQ + my_q,),
                device_id_type=pl.DeviceIdType.MESH,
            ).wait_recv()
        pl.semaphore_wait(ready_sem, NQ - 1)

        for off in range(1, NQ):
            tq = (my_q + off) % NQ
            pltpu.make_async_remote_copy(
                src_ref=recv_a.at[:, pl.ds(tq, 1)],
                dst_ref=recv_b.at[:, pl.ds(my_q, 1)],
                send_sem=send_b_sems.at[tq],
                recv_sem=recv_b_sems.at[my_q],
                device_id=(my_z * NQ + tq,),
                device_id_type=pl.DeviceIdType.MESH,
            ).start()
        recv_b[:, pl.ds(my_q, 1)] = recv_a[:, pl.ds(my_q, 1)]
        for off in range(1, NQ):
            sq = (my_q + off) % NQ
            pltpu.make_async_remote_copy(
                src_ref=recv_a.at[:, pl.ds(sq, 1)],
                dst_ref=recv_b.at[:, pl.ds(sq, 1)],
                send_sem=send_b_sems.at[sq],
                recv_sem=recv_b_sems.at[sq],
                device_id=(my_z * NQ + sq,),
                device_id_type=pl.DeviceIdType.MESH,
            ).wait_recv()

        xkt = recv_b[...].reshape(k_glob, M_BLK).astype(jnp.float32)
        w_copy.wait()
        y = lax.dot_general(xkt, w_vmem[...], (((0,), (0,)), ((), ())),
                            preferred_element_type=jnp.float32)
        y_vmem[...] = jnp.maximum(y, 0.0).astype(jnp.bfloat16)
        out_copy = pltpu.make_async_copy(y_vmem, out_ref, out_sem)
        out_copy.start()
        out_copy.wait()

        for off in range(1, NZ):
            tz = (my_z + off) % NZ
            pltpu.make_async_remote_copy(
                src_ref=send_buf.at[pl.ds(tz, 1)],
                dst_ref=recv_a.at[pl.ds(my_z, 1)],
                send_sem=send_a_sems.at[tz],
                recv_sem=recv_a_sems.at[my_z],
                device_id=(tz * NQ + my_q,),
                device_id_type=pl.DeviceIdType.MESH,
            ).wait_send()
        for off in range(1, NQ):
            tq = (my_q + off) % NQ
            pltpu.make_async_remote_copy(
                src_ref=recv_a.at[:, pl.ds(tq, 1)],
                dst_ref=recv_b.at[:, pl.ds(my_q, 1)],
                send_sem=send_b_sems.at[tq],
                recv_sem=recv_b_sems.at[my_q],
                device_id=(my_z * NQ + tq,),
                device_id_type=pl.DeviceIdType.MESH,
            ).wait_send()

    return pl.pallas_call(
        body,
        out_shape=jax.ShapeDtypeStruct((M_BLK, n), jnp.bfloat16),
        in_specs=[
            pl.BlockSpec(memory_space=pltpu.MemorySpace.HBM),
            pl.BlockSpec(memory_space=pltpu.MemorySpace.HBM),
        ],
        out_specs=pl.BlockSpec(memory_space=pltpu.MemorySpace.HBM),
        scratch_shapes=[
            pltpu.VMEM((NZ, NQ, K_BLK, M_BLK), jnp.bfloat16),
            pltpu.VMEM((NZ, NQ, K_BLK, M_BLK), jnp.bfloat16),
            pltpu.VMEM((NZ, NQ, K_BLK, M_BLK), jnp.bfloat16),
            pltpu.VMEM((1024, 1024), jnp.float32),
            pltpu.VMEM((K_BLK, 1024), jnp.float32),
            pltpu.VMEM((M_BLK, 1024), jnp.bfloat16),
            pltpu.SemaphoreType.DMA((NZ,)),
            pltpu.SemaphoreType.DMA((NZ,)),
            pltpu.SemaphoreType.DMA((NQ,)),
            pltpu.SemaphoreType.DMA((NQ,)),
            pltpu.SemaphoreType.DMA,
            pltpu.SemaphoreType.DMA,
            pltpu.SemaphoreType.DMA,
            pltpu.SemaphoreType.REGULAR,
        ],
        compiler_params=pltpu.CompilerParams(collective_id=0),
    )(xt, w_hbm)
